# baseline (speedup 1.0000x reference)
import functools

import jax
import jax.numpy as jnp
from jax import lax
from jax.experimental import pallas as pl
from jax.experimental.pallas import tpu as pltpu
from jax.experimental.pallas import tpu_sc as plsc

K = 2048


def _rownorm(x):
    p = x[:, 0:128] * x[:, 0:128]
    for c in range(1, 4):
        xc = x[:, 128 * c:128 * (c + 1)]
        p = p + xc * xc
    pt = jnp.transpose(p)
    w = pt[0:8]
    for m in range(1, 16):
        w = w + pt[8 * m:8 * (m + 1)]
    t1 = w[0:4] + w[4:8]
    t2 = t1[0:2] + t1[2:4]
    t3 = t2[0:1] + t2[1:2]
    return jnp.sqrt(t3)


def _norm_body(se_ref, te_ref, sn_ref, tn_ref):
    xs = se_ref[0]
    xt = te_ref[0]
    sn_ref[...] = _rownorm(xs)[None]
    tn_ref[...] = _rownorm(xt)[None]


def _norms(src_embedding, tgt_embedding):
    B, N, D = src_embedding.shape
    CH = 2048
    nch = N // CH
    grid = (B, nch)
    sn, tn = pl.pallas_call(
        _norm_body,
        grid=grid,
        in_specs=[
            pl.BlockSpec((1, CH, D), lambda b, c: (b, c, 0)),
            pl.BlockSpec((1, CH, D), lambda b, c: (b, c, 0)),
        ],
        out_specs=[
            pl.BlockSpec((1, 1, CH), lambda b, c: (b * nch + c, 0, 0)),
            pl.BlockSpec((1, 1, CH), lambda b, c: (b * nch + c, 0, 0)),
        ],
        out_shape=[
            jax.ShapeDtypeStruct((B * nch, 1, CH), jnp.float32),
            jax.ShapeDtypeStruct((B * nch, 1, CH), jnp.float32),
        ],
    )(src_embedding, tgt_embedding)
    return sn.reshape(B, N), tn.reshape(B, N)


def _make_gather(BN, BK, D):
    info = plsc.get_sparse_core_info()
    NC, NS = info.num_cores, info.num_subcores
    NW = NC * NS
    RPW = BK // NW
    ECH = 32
    SCH = 128
    mesh = plsc.VectorSubcoreMesh(core_axis_name="c", subcore_axis_name="s")

    @functools.partial(
        pl.kernel, mesh=mesh,
        out_type=[
            jax.ShapeDtypeStruct((BK, 128), jnp.float32),
            jax.ShapeDtypeStruct((BK, 128), jnp.float32),
            jax.ShapeDtypeStruct((BK, D), jnp.float32),
            jax.ShapeDtypeStruct((BK, D), jnp.float32),
        ],
        scratch_types=[
            pltpu.VMEM((RPW,), jnp.int32),
            pltpu.VMEM((RPW,), jnp.int32),
            pltpu.VMEM((SCH, 128), jnp.float32),
            pltpu.VMEM((ECH, D), jnp.float32),
            pltpu.VMEM((ECH, D), jnp.float32),
            pltpu.SemaphoreType.DMA,
            pltpu.SemaphoreType.DMA,
        ],
    )
    def g(pts, se, te, sidx, tidx,
          sp_out, tp_out, se_out, te_out,
          sidx_v, tidx_v, small_buf, ebuf0, ebuf1, sem0, sem1):
        wid = lax.axis_index("s") * NC + lax.axis_index("c")
        base = wid * RPW
        pltpu.sync_copy(sidx.at[pl.ds(base, RPW)], sidx_v)
        pltpu.sync_copy(tidx.at[pl.ds(base, RPW)], tidx_v)
        for side in range(2):
            out = sp_out if side == 0 else tp_out
            idx = sidx_v if side == 0 else tidx_v
            for c in range(RPW // SCH):
                pltpu.async_copy(
                    pts.at[idx.at[pl.ds(c * SCH, SCH)]], small_buf,
                    sem0).wait()
                pltpu.sync_copy(
                    small_buf, out.at[pl.ds(base + c * SCH, SCH)])
        nch = RPW // ECH
        for side in range(2):
            tab = se if side == 0 else te
            out = se_out if side == 0 else te_out
            idx = sidx_v if side == 0 else tidx_v
            for c in range(nch):
                buf = ebuf0 if c % 2 == 0 else ebuf1
                sem = sem0 if c % 2 == 0 else sem1
                pltpu.async_copy(
                    tab.at[idx.at[pl.ds(c * ECH, ECH)]], buf, sem).wait()
                pltpu.sync_copy(
                    buf, out.at[pl.ds(base + c * ECH, ECH)])

    return g


def kernel(src, tgt, n0, n1, src_embedding, tgt_embedding):
    B, N, D = src_embedding.shape
    src_norm, tgt_norm = _norms(src_embedding, tgt_embedding)
    _, src_idx = jax.lax.top_k(src_norm, K)
    _, tgt_idx = jax.lax.top_k(tgt_norm, K)
    off = (jnp.arange(B, dtype=jnp.int32) * N)[:, None]
    sidx = (src_idx + off).reshape(-1)
    tidx = (tgt_idx + off).reshape(-1)
    zpad = jnp.zeros((B, N, 116), jnp.float32)
    pts = jnp.concatenate([src, n0, tgt, n1, zpad], axis=-1).reshape(B * N, 128)
    g = _make_gather(B * N, B * K, D)
    sp, tp, se_k, te_k = g(pts,
                           src_embedding.reshape(B * N, D),
                           tgt_embedding.reshape(B * N, D),
                           sidx, tidx)
    sp = sp.reshape(B, K, 128)
    tp = tp.reshape(B, K, 128)
    return (sp[:, :, 0:3], tp[:, :, 6:9],
            sp[:, :, 3:6], tp[:, :, 9:12],
            se_k.reshape(B, K, D), te_k.reshape(B, K, D))
